# Initial kernel scaffold; baseline (speedup 1.0000x reference)
import functools

import jax
import jax.numpy as jnp
from jax import lax
from jax.experimental import pallas as pl
from jax.experimental.pallas import tpu as pltpu
from jax.experimental.pallas import tpu_sc as plsc

B, S = 1024, 200
VOCAB, DIM = 100000, 128
TAG_VOCAB, TAG_DIM = 60, 4
OUT_DIM = DIM + TAG_DIM + 1

_info = plsc.get_sparse_core_info()
NC, NS, L = _info.num_cores, _info.num_subcores, _info.num_lanes
NW = NC * NS

R = S * B
RPW = R // NW
C = 128
NCHUNK = RPW // C

_mesh = plsc.VectorSubcoreMesh(core_axis_name="c", subcore_axis_name="s")


@functools.partial(
    pl.kernel,
    mesh=_mesh,
    out_type=jax.ShapeDtypeStruct((R, OUT_DIM), jnp.float32),
    scratch_types=[
        pltpu.VMEM((C,), jnp.int32),
        pltpu.VMEM((C,), jnp.int32),
        pltpu.VMEM((C,), jnp.int32),
        pltpu.VMEM((C, OUT_DIM), jnp.float32),
        pltpu.VMEM((TAG_VOCAB, TAG_DIM), jnp.float32),
        pltpu.SemaphoreType.DMA,
    ],
)
def _extend_embedding_sc(widx_hbm, tidx_hbm, isin_hbm, wtab_hbm, ttab_hbm,
                         out_hbm, widx_v, tidx_v, isin_v, out_v, ttab_v, sem):
    wid = lax.axis_index("s") * NC + lax.axis_index("c")
    pltpu.sync_copy(ttab_hbm, ttab_v)
    base0 = wid * RPW

    def chunk_body(ci, _):
        base = base0 + ci * C
        pltpu.sync_copy(widx_hbm.at[pl.ds(base, C)], widx_v)
        pltpu.sync_copy(tidx_hbm.at[pl.ds(base, C)], tidx_v)
        pltpu.sync_copy(isin_hbm.at[pl.ds(base, C)], isin_v)
        pltpu.async_copy(wtab_hbm.at[widx_v], out_v, sem).wait()

        def grp_body(g, _):
            r0 = g * L
            rows = r0 + lax.iota(jnp.int32, L)
            t = tidx_v[pl.ds(r0, L)]
            for c4 in range(TAG_DIM):
                col = jnp.full((L,), DIM + c4, jnp.int32)
                vals = plsc.load_gather(ttab_v, [t, jnp.full((L,), c4, jnp.int32)])
                plsc.store_scatter(out_v, [rows, col], vals)
            ii = isin_v[pl.ds(r0, L)].astype(jnp.float32)
            plsc.store_scatter(out_v, [rows, jnp.full((L,), DIM + TAG_DIM, jnp.int32)], ii)
            return 0

        lax.fori_loop(0, C // L, grp_body, 0)
        pltpu.sync_copy(out_v, out_hbm.at[pl.ds(base, C)])
        return 0

    lax.fori_loop(0, NCHUNK, chunk_body, 0)


def kernel(word_ids, tag_ids, is_in, word_table, tag_table):
    widx = jnp.swapaxes(word_ids, 0, 1).reshape(R)
    tidx = jnp.swapaxes(tag_ids, 0, 1).reshape(R)
    iidx = jnp.swapaxes(is_in, 0, 1).reshape(R)
    wpad = jnp.pad(word_table, ((0, 0), (0, OUT_DIM - DIM)))
    out = _extend_embedding_sc(widx, tidx, iidx, wpad, tag_table)
    return out.reshape(S, B, OUT_DIM)


# speedup vs baseline: 2.0252x; 2.0252x over previous
import functools

import jax
import jax.numpy as jnp
from jax import lax
from jax.experimental import pallas as pl
from jax.experimental.pallas import tpu as pltpu
from jax.experimental.pallas import tpu_sc as plsc

B, S = 1024, 200
VOCAB, DIM = 100000, 128
TAG_VOCAB, TAG_DIM = 60, 4
OUT_DIM = DIM + TAG_DIM + 1

_info = plsc.get_sparse_core_info()
NC, NS, L = _info.num_cores, _info.num_subcores, _info.num_lanes
NW = NC * NS

R = S * B
RPW = R // NW
C = 128
NCHUNK = RPW // C

_mesh = plsc.VectorSubcoreMesh(core_axis_name="c", subcore_axis_name="s")


@functools.partial(
    pl.kernel,
    mesh=_mesh,
    out_type=jax.ShapeDtypeStruct((R, OUT_DIM), jnp.float32),
    compiler_params=pltpu.CompilerParams(
        needs_layout_passes=False, use_tc_tiling_on_sc=False),
    scratch_types=[
        pltpu.VMEM((C,), jnp.int32),
        pltpu.VMEM((C,), jnp.int32),
        pltpu.VMEM((C,), jnp.int32),
        pltpu.VMEM((C, OUT_DIM), jnp.float32),
        pltpu.VMEM((TAG_VOCAB * TAG_DIM,), jnp.float32),
        pltpu.SemaphoreType.DMA,
    ],
)
def _extend_embedding_sc(widx_hbm, tidx_hbm, isin_hbm, wtab_hbm, ttab_hbm,
                         out_hbm, widx_v, tidx_v, isin_v, out_v, ttab_v, sem):
    wid = lax.axis_index("s") * NC + lax.axis_index("c")
    pltpu.sync_copy(ttab_hbm, ttab_v)
    base0 = wid * RPW

    def chunk_body(ci, _):
        base = base0 + ci * C
        pltpu.sync_copy(widx_hbm.at[pl.ds(base, C)], widx_v)
        pltpu.sync_copy(tidx_hbm.at[pl.ds(base, C)], tidx_v)
        pltpu.sync_copy(isin_hbm.at[pl.ds(base, C)], isin_v)
        pltpu.async_copy(wtab_hbm.at[widx_v], out_v, sem).wait()

        def grp_body(g, _):
            r0 = g * L
            rows = r0 + lax.iota(jnp.int32, L)
            t = tidx_v[pl.ds(r0, L)]
            t4 = t * TAG_DIM
            for c4 in range(TAG_DIM):
                col = jnp.full((L,), DIM + c4, jnp.int32)
                vals = plsc.load_gather(ttab_v, [t4 + c4])
                plsc.store_scatter(out_v, [rows, col], vals)
            ii = isin_v[pl.ds(r0, L)].astype(jnp.float32)
            plsc.store_scatter(out_v, [rows, jnp.full((L,), DIM + TAG_DIM, jnp.int32)], ii)
            return 0

        lax.fori_loop(0, C // L, grp_body, 0)
        pltpu.sync_copy(out_v, out_hbm.at[pl.ds(base, C)])
        return 0

    lax.fori_loop(0, NCHUNK, chunk_body, 0)


def kernel(word_ids, tag_ids, is_in, word_table, tag_table):
    widx = jnp.swapaxes(word_ids, 0, 1).reshape(R)
    tidx = jnp.swapaxes(tag_ids, 0, 1).reshape(R)
    iidx = jnp.swapaxes(is_in, 0, 1).reshape(R)
    wpad = jnp.pad(word_table, ((0, 0), (0, OUT_DIM - DIM)))
    out = _extend_embedding_sc(widx, tidx, iidx, wpad, tag_table.reshape(-1))
    return out.reshape(S, B, OUT_DIM)


# speedup vs baseline: 5.8816x; 2.9042x over previous
import functools

import jax
import jax.numpy as jnp
from jax import lax
from jax.experimental import pallas as pl
from jax.experimental.pallas import tpu as pltpu
from jax.experimental.pallas import tpu_sc as plsc

B, S = 1024, 200
VOCAB, DIM = 100000, 128
TAG_VOCAB, TAG_DIM = 60, 4
OUT_DIM = DIM + TAG_DIM + 1
TAIL = OUT_DIM - DIM

_info = plsc.get_sparse_core_info()
NC, NS, L = _info.num_cores, _info.num_subcores, _info.num_lanes
NW = NC * NS

R = S * B
RPW = R // NW
C = 64
NCHUNK = RPW // C
NB = 4
NT = 2
LOOKAHEAD = 2

_mesh = plsc.VectorSubcoreMesh(core_axis_name="c", subcore_axis_name="s")


@functools.partial(
    pl.kernel,
    mesh=_mesh,
    out_type=jax.ShapeDtypeStruct((R, OUT_DIM), jnp.float32),
    compiler_params=pltpu.CompilerParams(needs_layout_passes=False),
    scratch_types=[
        pltpu.VMEM((RPW,), jnp.int32),
        pltpu.VMEM((RPW,), jnp.int32),
        pltpu.VMEM((RPW,), jnp.int32),
        pltpu.VMEM((NB, C, DIM), jnp.float32),
        pltpu.VMEM((NT, C, TAIL), jnp.float32),
        pltpu.VMEM((TAG_VOCAB * TAG_DIM,), jnp.float32),
        pltpu.SemaphoreType.DMA,
        pltpu.SemaphoreType.DMA,
        pltpu.SemaphoreType.DMA,
        pltpu.SemaphoreType.DMA,
        pltpu.SemaphoreType.DMA,
        pltpu.SemaphoreType.DMA,
        pltpu.SemaphoreType.DMA,
        pltpu.SemaphoreType.DMA,
        pltpu.SemaphoreType.DMA,
        pltpu.SemaphoreType.DMA,
    ],
)
def _extend_embedding_sc(widx_hbm, tidx_hbm, isin_hbm, wtab_hbm, ttab_hbm,
                         out_hbm, widx_v, tidx_v, isin_v, word_v, tail_v,
                         ttab_v, *sems):
    sem_g = sems[0:NB]
    sem_w = sems[NB:2 * NB]
    sem_t = sems[2 * NB:2 * NB + NT]
    wid = lax.axis_index("s") * NC + lax.axis_index("c")
    slab = wid * RPW
    pltpu.sync_copy(ttab_hbm, ttab_v)
    pltpu.sync_copy(widx_hbm.at[pl.ds(slab, RPW)], widx_v)
    pltpu.sync_copy(tidx_hbm.at[pl.ds(slab, RPW)], tidx_v)
    pltpu.sync_copy(isin_hbm.at[pl.ds(slab, RPW)], isin_v)

    def gather_chunk(j, b):
        pltpu.async_copy(
            wtab_hbm.at[widx_v.at[pl.ds(j * C, C)]], word_v.at[b], sem_g[b])

    for b in range(LOOKAHEAD):
        gather_chunk(b, b)

    def outer(g, _):
        for b in range(NB):
            j = g * NB + b
            bt = b % NT
            base = slab + j * C
            out_word = out_hbm.at[pl.ds(base, C), pl.ds(0, DIM)]
            out_tail = out_hbm.at[pl.ds(base, C), pl.ds(DIM, TAIL)]
            pltpu.make_async_copy(
                wtab_hbm.at[widx_v.at[pl.ds(j * C, C)]], word_v.at[b],
                sem_g[b]).wait()
            @pl.when(jnp.logical_or(g > 0, b >= NT))
            def _():
                pltpu.make_async_copy(tail_v.at[bt], out_tail, sem_t[bt]).wait()

            def grp(k, _):
                r0 = j * C + k * L
                rows = k * L + lax.iota(jnp.int32, L)
                t4 = tidx_v[pl.ds(r0, L)] * TAG_DIM
                bb = jnp.full((L,), b % NT, jnp.int32)
                for c4 in range(TAG_DIM):
                    vals = plsc.load_gather(ttab_v, [t4 + c4])
                    plsc.store_scatter(
                        tail_v, [bb, rows, jnp.full((L,), c4, jnp.int32)], vals)
                ii = isin_v[pl.ds(r0, L)].astype(jnp.float32)
                plsc.store_scatter(
                    tail_v, [bb, rows, jnp.full((L,), TAG_DIM, jnp.int32)], ii)
                return 0

            lax.fori_loop(0, C // L, grp, 0)
            pltpu.async_copy(word_v.at[b], out_word, sem_w[b])
            pltpu.async_copy(tail_v.at[bt], out_tail, sem_t[bt])
            jn = j + LOOKAHEAD
            b2 = (b + LOOKAHEAD) % NB

            @pl.when(jn >= NB)
            def _():
                basep = slab + (jn - NB) * C
                pltpu.make_async_copy(
                    word_v.at[b2],
                    out_hbm.at[pl.ds(basep, C), pl.ds(0, DIM)],
                    sem_w[b2]).wait()

            @pl.when(jn < NCHUNK)
            def _():
                gather_chunk(jn, b2)
        return 0

    lax.fori_loop(0, NCHUNK // NB, outer, 0)
    for j in range(NCHUNK - NB + LOOKAHEAD, NCHUNK):
        b2 = j % NB
        base = slab + j * C
        pltpu.make_async_copy(
            word_v.at[b2], out_hbm.at[pl.ds(base, C), pl.ds(0, DIM)],
            sem_w[b2]).wait()
    for j in range(NCHUNK - NT, NCHUNK):
        bt = j % NT
        base = slab + j * C
        pltpu.make_async_copy(
            tail_v.at[bt], out_hbm.at[pl.ds(base, C), pl.ds(DIM, TAIL)],
            sem_t[bt]).wait()


def kernel(word_ids, tag_ids, is_in, word_table, tag_table):
    widx = jnp.swapaxes(word_ids, 0, 1).reshape(R)
    tidx = jnp.swapaxes(tag_ids, 0, 1).reshape(R)
    iidx = jnp.swapaxes(is_in, 0, 1).reshape(R)
    out = _extend_embedding_sc(widx, tidx, iidx, word_table,
                               tag_table.reshape(-1))
    return out.reshape(S, B, OUT_DIM)
